# Initial kernel scaffold; baseline (speedup 1.0000x reference)
import jax
import jax.numpy as jnp
from jax.experimental import pallas as pl


def kernel(feat, edge_index, W1, b1, W2, b2):
    raise NotImplementedError("write your pallas kernel here")


# speedup vs baseline: 3.3965x; 3.3965x over previous
import functools

import jax
import jax.numpy as jnp
from jax import lax
from jax.experimental import pallas as pl
from jax.experimental.pallas import tpu as pltpu
from jax.experimental.pallas import tpu_sc as plsc

N_NODES = 10000
N_EDGES = 320000
NP = 10240
EP = 327680
D = 128
HID = 256
NC = 2
NS = 16
NW = NC * NS
C = 128
CH_W = EP // NW // C
CH_T = EP // NS // C
ROWS_T = NP // NS

_mesh = plsc.VectorSubcoreMesh(core_axis_name="c", subcore_axis_name="s")


def _zero_vmem(buf, rows, cols):
    zero = jnp.zeros((16,), buf.dtype)

    @pl.loop(0, rows)
    def _(i):
        for k in range(cols // 16):
            buf[i, pl.ds(k * 16, 16)] = zero


_IDENT_CH = ROWS_T // C


@functools.partial(
    pl.kernel,
    out_type=jax.ShapeDtypeStruct((NC, NP, D), jnp.float32),
    mesh=_mesh,
    scratch_types=[
        pltpu.VMEM((CH_W, C), jnp.int32),
        pltpu.VMEM((C, D), jnp.float32),
        pltpu.VMEM((C, D), jnp.float32),
        pltpu.VMEM((_IDENT_CH, C), jnp.int32),
        pltpu.VMEM_SHARED((NP, D), jnp.float32),
        pltpu.SemaphoreType.DMA,
    ],
)
def _count(idx_hbm, ident_hbm, out_hbm, idx_v, ones_v, zbuf, ident_v, hist, sem):
    c = lax.axis_index("c")
    s = lax.axis_index("s")
    pltpu.sync_copy(idx_hbm.at[s * NC + c], idx_v)
    pltpu.sync_copy(ident_hbm.at[s], ident_v)

    one = jnp.ones((16,), jnp.float32)

    @pl.loop(0, C)
    def _(i):
        for k in range(D // 16):
            ones_v[i, pl.ds(k * 16, 16)] = one

    _zero_vmem(zbuf, C, D)
    for j in range(_IDENT_CH):
        pltpu.sync_copy(zbuf, hist.at[ident_v.at[j]])
    plsc.subcore_barrier()

    @pl.loop(0, CH_W)
    def _(j):
        pltpu.sync_copy(ones_v, hist.at[idx_v.at[j]], add=True)

    plsc.subcore_barrier()
    for j in range(_IDENT_CH):
        pltpu.async_copy(hist.at[ident_v.at[j]], zbuf, sem).wait()
        pltpu.sync_copy(zbuf, out_hbm.at[c, pl.ds(s * ROWS_T + j * C, C)])


@functools.partial(
    pl.kernel,
    out_type=jax.ShapeDtypeStruct((NC, NP, D), jnp.float32),
    mesh=_mesh,
    scratch_types=[
        pltpu.VMEM((CH_W, C), jnp.int32),
        pltpu.VMEM((CH_W, C), jnp.int32),
        pltpu.VMEM((C, D), jnp.float32),
        pltpu.VMEM((_IDENT_CH, C), jnp.int32),
        pltpu.VMEM_SHARED((NP, D), jnp.float32),
        pltpu.SemaphoreType.DMA,
    ],
)
def _agg(x_hbm, src_hbm, dst_hbm, ident_hbm, out_hbm, src_v, dst_v, gbuf, ident_v, acc, sem):
    c = lax.axis_index("c")
    s = lax.axis_index("s")
    wid = s * NC + c
    pltpu.sync_copy(src_hbm.at[wid], src_v)
    pltpu.sync_copy(dst_hbm.at[wid], dst_v)
    pltpu.sync_copy(ident_hbm.at[s], ident_v)

    _zero_vmem(gbuf, C, D)
    for j in range(_IDENT_CH):
        pltpu.sync_copy(gbuf, acc.at[ident_v.at[j]])
    plsc.subcore_barrier()

    @pl.loop(0, CH_W)
    def _(j):
        pltpu.async_copy(x_hbm.at[src_v.at[j]], gbuf, sem).wait()
        pltpu.sync_copy(gbuf, acc.at[dst_v.at[j]], add=True)

    plsc.subcore_barrier()
    for j in range(_IDENT_CH):
        pltpu.async_copy(acc.at[ident_v.at[j]], gbuf, sem).wait()
        pltpu.sync_copy(gbuf, out_hbm.at[c, pl.ds(s * ROWS_T + j * C, C)])


_BLK = 1280
_GRID = NP // _BLK


def _norm(deg):
    return jnp.where(deg > 0, lax.rsqrt(deg), 1.0)


def _scale_body(deg_out_ref, feat_ref, x1_ref):
    ns = _norm(deg_out_ref[:, 0])
    x1_ref[:, :] = feat_ref[:, :] * ns[:, None]


def _scale(deg_out, featp):
    return pl.pallas_call(
        _scale_body,
        grid=(_GRID,),
        in_specs=[
            pl.BlockSpec((_BLK, 1), lambda i: (i, 0)),
            pl.BlockSpec((_BLK, D), lambda i: (i, 0)),
        ],
        out_specs=pl.BlockSpec((_BLK, D), lambda i: (i, 0)),
        out_shape=jax.ShapeDtypeStruct((NP, D), jnp.float32),
    )(deg_out, featp)


def _mid_body(p_ref, do_ref, di_ref, w1_ref, b1_ref, w2_ref, x2_ref):
    i = pl.program_id(0)
    a1 = p_ref[0, :, :] + p_ref[1, :, :]
    nd = _norm(di_ref[:, 0])
    ns = _norm(do_ref[:, 0])
    h = jnp.dot(a1, w1_ref[:, :], preferred_element_type=jnp.float32)
    h = jnp.maximum(h * nd[:, None] + b1_ref[:][None, :], 0.0)
    x2 = jnp.dot(h * ns[:, None], w2_ref[:, :], preferred_element_type=jnp.float32)
    row = i * _BLK + lax.broadcasted_iota(jnp.int32, (_BLK, 1), 0)
    x2_ref[:, :] = jnp.where(row < N_NODES, x2, 0.0)


def _mid(p1, deg_out, deg_in, W1, b1, W2):
    return pl.pallas_call(
        _mid_body,
        grid=(_GRID,),
        in_specs=[
            pl.BlockSpec((NC, _BLK, D), lambda i: (0, i, 0)),
            pl.BlockSpec((_BLK, 1), lambda i: (i, 0)),
            pl.BlockSpec((_BLK, 1), lambda i: (i, 0)),
            pl.BlockSpec((D, HID), lambda i: (0, 0)),
            pl.BlockSpec((HID,), lambda i: (0,)),
            pl.BlockSpec((HID, D), lambda i: (0, 0)),
        ],
        out_specs=pl.BlockSpec((_BLK, D), lambda i: (i, 0)),
        out_shape=jax.ShapeDtypeStruct((NP, D), jnp.float32),
    )(p1, deg_out, deg_in, W1, b1, W2)


def _final_body(p_ref, di_ref, b2_ref, out_ref):
    nd = _norm(di_ref[:, 0])
    a2 = p_ref[0, :, :] + p_ref[1, :, :]
    out_ref[:, :] = a2 * nd[:, None] + b2_ref[:][None, :]


def _final(p2, deg_in, b2):
    return pl.pallas_call(
        _final_body,
        grid=(_GRID,),
        in_specs=[
            pl.BlockSpec((NC, _BLK, D), lambda i: (0, i, 0)),
            pl.BlockSpec((_BLK, 1), lambda i: (i, 0)),
            pl.BlockSpec((D,), lambda i: (0,)),
        ],
        out_specs=pl.BlockSpec((_BLK, D), lambda i: (i, 0)),
        out_shape=jax.ShapeDtypeStruct((NP, D), jnp.float32),
    )(p2, deg_in, b2)


def kernel(feat, edge_index, W1, b1, W2, b2):
    ei = edge_index.astype(jnp.int32)
    pad = jnp.full((EP - N_EDGES,), N_NODES, jnp.int32)
    srcp = jnp.concatenate([ei[0], pad])
    dstp = jnp.concatenate([ei[1], pad])
    src_w = srcp.reshape(NW, CH_W, C)
    dst_w = dstp.reshape(NW, CH_W, C)
    featp = jnp.pad(feat, ((0, NP - N_NODES), (0, 0)))

    ident = jnp.arange(NP, dtype=jnp.int32).reshape(NS, _IDENT_CH, C)

    cnt_s = _count(src_w, ident)
    cnt_d = _count(dst_w, ident)
    deg_out = (cnt_s[0, :, 0] + cnt_s[1, :, 0]).reshape(NP, 1)
    deg_in = (cnt_d[0, :, 0] + cnt_d[1, :, 0]).reshape(NP, 1)

    x1 = _scale(deg_out, featp)
    p1 = _agg(x1, src_w, dst_w, ident)
    x2 = _mid(p1, deg_out, deg_in, W1, b1, W2)
    p2 = _agg(x2, src_w, dst_w, ident)
    outp = _final(p2, deg_in, b2)
    return outp[:N_NODES]


# speedup vs baseline: 3.6139x; 1.0640x over previous
import functools

import jax
import jax.numpy as jnp
from jax import lax
from jax.experimental import pallas as pl
from jax.experimental.pallas import tpu as pltpu
from jax.experimental.pallas import tpu_sc as plsc

N_NODES = 10000
N_EDGES = 320000
NP = 10240
EP = 327680
D = 128
HID = 256
NC = 2
NS = 16
NW = NC * NS
C = 128
CH_W = EP // NW // C
CH_T = EP // NS // C
ROWS_T = NP // NS

_mesh = plsc.VectorSubcoreMesh(core_axis_name="c", subcore_axis_name="s")


def _zero_vmem(buf, rows, cols):
    zero = jnp.zeros((16,), buf.dtype)

    @pl.loop(0, rows)
    def _(i):
        for k in range(cols // 16):
            buf[i, pl.ds(k * 16, 16)] = zero


_IDENT_CH = ROWS_T // C


@functools.partial(
    pl.kernel,
    out_type=jax.ShapeDtypeStruct((NC, NP, D), jnp.float32),
    mesh=_mesh,
    scratch_types=[
        pltpu.VMEM((CH_W, C), jnp.int32),
        pltpu.VMEM((C, D), jnp.float32),
        pltpu.VMEM((C, D), jnp.float32),
        pltpu.VMEM((_IDENT_CH, C), jnp.int32),
        pltpu.VMEM_SHARED((NP, D), jnp.float32),
        pltpu.SemaphoreType.DMA,
    ],
)
def _count(idx_hbm, ident_hbm, out_hbm, idx_v, ones_v, zbuf, ident_v, hist, sem):
    c = lax.axis_index("c")
    s = lax.axis_index("s")
    pltpu.sync_copy(idx_hbm.at[s * NC + c], idx_v)
    pltpu.sync_copy(ident_hbm.at[s], ident_v)

    one = jnp.ones((16,), jnp.float32)

    @pl.loop(0, C)
    def _(i):
        for k in range(D // 16):
            ones_v[i, pl.ds(k * 16, 16)] = one

    _zero_vmem(zbuf, C, D)
    for j in range(_IDENT_CH):
        pltpu.sync_copy(zbuf, hist.at[ident_v.at[j]])
    plsc.subcore_barrier()

    @pl.loop(0, CH_W)
    def _(j):
        pltpu.sync_copy(ones_v, hist.at[idx_v.at[j]], add=True)

    plsc.subcore_barrier()
    for j in range(_IDENT_CH):
        pltpu.async_copy(hist.at[ident_v.at[j]], zbuf, sem).wait()
        pltpu.sync_copy(zbuf, out_hbm.at[c, pl.ds(s * ROWS_T + j * C, C)])


@functools.partial(
    pl.kernel,
    out_type=jax.ShapeDtypeStruct((NC, NP, D), jnp.float32),
    mesh=_mesh,
    scratch_types=[
        pltpu.VMEM((CH_W // 2, C), jnp.int32),
        pltpu.VMEM((CH_W // 2, C), jnp.int32),
        pltpu.VMEM((C, D), jnp.float32),
        pltpu.VMEM((C, D), jnp.float32),
        pltpu.VMEM((_IDENT_CH, C), jnp.int32),
        pltpu.VMEM_SHARED((NP, D), jnp.float32),
        pltpu.SemaphoreType.DMA,
        pltpu.SemaphoreType.DMA,
    ],
)
def _agg(x_hbm, src_hbm, dst_hbm, ident_hbm, out_hbm,
         src_v, dst_v, gb0, gb1, ident_v, acc, gsem, ssem):
    c = lax.axis_index("c")
    s = lax.axis_index("s")
    wid = s * NC + c
    pltpu.sync_copy(ident_hbm.at[s], ident_v)

    _zero_vmem(gb0, C, D)
    for j in range(_IDENT_CH):
        pltpu.sync_copy(gb0, acc.at[ident_v.at[j]])
    plsc.subcore_barrier()

    def g_start(j, buf):
        pltpu.async_copy(x_hbm.at[src_v.at[j]], buf, gsem)

    def g_wait(j, buf):
        pltpu.make_async_copy(x_hbm.at[src_v.at[j]], buf, gsem).wait()

    def s_start(j, buf):
        pltpu.async_copy(buf, acc.at[dst_v.at[j]], ssem, add=True)

    def s_wait(j, buf):
        pltpu.make_async_copy(buf, acc.at[dst_v.at[j]], ssem).wait()

    CH_H = CH_W // 2
    for h in range(2):
        pltpu.sync_copy(src_hbm.at[wid * 2 + h], src_v)
        pltpu.sync_copy(dst_hbm.at[wid * 2 + h], dst_v)

        g_start(0, gb0)
        g_wait(0, gb0)
        g_start(1, gb1)
        s_start(0, gb0)
        g_wait(1, gb1)
        s_wait(0, gb0)
        g_start(2, gb0)
        s_start(1, gb1)

        @pl.loop(1, CH_H // 2 - 1)
        def _(g):
            j0 = g * 2
            g_wait(j0, gb0)
            s_wait(j0 - 1, gb1)
            g_start(j0 + 1, gb1)
            s_start(j0, gb0)
            g_wait(j0 + 1, gb1)
            s_wait(j0, gb0)
            g_start(j0 + 2, gb0)
            s_start(j0 + 1, gb1)

        jl = CH_H - 2
        g_wait(jl, gb0)
        s_wait(jl - 1, gb1)
        g_start(jl + 1, gb1)
        s_start(jl, gb0)
        g_wait(jl + 1, gb1)
        s_wait(jl, gb0)
        s_start(jl + 1, gb1)
        s_wait(jl + 1, gb1)

    plsc.subcore_barrier()
    for j in range(_IDENT_CH):
        pltpu.async_copy(acc.at[ident_v.at[j]], gb0, gsem).wait()
        pltpu.sync_copy(gb0, out_hbm.at[c, pl.ds(s * ROWS_T + j * C, C)])


_BLK = 1280
_GRID = NP // _BLK


def _norm(deg):
    return jnp.where(deg > 0, lax.rsqrt(deg), 1.0)


def _scale_body(deg_out_ref, feat_ref, x1_ref):
    ns = _norm(deg_out_ref[:, 0])
    x1_ref[:, :] = feat_ref[:, :] * ns[:, None]


def _scale(deg_out, featp):
    return pl.pallas_call(
        _scale_body,
        grid=(_GRID,),
        in_specs=[
            pl.BlockSpec((_BLK, 1), lambda i: (i, 0)),
            pl.BlockSpec((_BLK, D), lambda i: (i, 0)),
        ],
        out_specs=pl.BlockSpec((_BLK, D), lambda i: (i, 0)),
        out_shape=jax.ShapeDtypeStruct((NP, D), jnp.float32),
    )(deg_out, featp)


def _mid_body(p_ref, do_ref, di_ref, w1_ref, b1_ref, w2_ref, x2_ref):
    i = pl.program_id(0)
    a1 = p_ref[0, :, :] + p_ref[1, :, :]
    nd = _norm(di_ref[:, 0])
    ns = _norm(do_ref[:, 0])
    h = jnp.dot(a1, w1_ref[:, :], preferred_element_type=jnp.float32)
    h = jnp.maximum(h * nd[:, None] + b1_ref[:][None, :], 0.0)
    x2 = jnp.dot(h * ns[:, None], w2_ref[:, :], preferred_element_type=jnp.float32)
    row = i * _BLK + lax.broadcasted_iota(jnp.int32, (_BLK, 1), 0)
    x2_ref[:, :] = jnp.where(row < N_NODES, x2, 0.0)


def _mid(p1, deg_out, deg_in, W1, b1, W2):
    return pl.pallas_call(
        _mid_body,
        grid=(_GRID,),
        in_specs=[
            pl.BlockSpec((NC, _BLK, D), lambda i: (0, i, 0)),
            pl.BlockSpec((_BLK, 1), lambda i: (i, 0)),
            pl.BlockSpec((_BLK, 1), lambda i: (i, 0)),
            pl.BlockSpec((D, HID), lambda i: (0, 0)),
            pl.BlockSpec((HID,), lambda i: (0,)),
            pl.BlockSpec((HID, D), lambda i: (0, 0)),
        ],
        out_specs=pl.BlockSpec((_BLK, D), lambda i: (i, 0)),
        out_shape=jax.ShapeDtypeStruct((NP, D), jnp.float32),
    )(p1, deg_out, deg_in, W1, b1, W2)


def _final_body(p_ref, di_ref, b2_ref, out_ref):
    nd = _norm(di_ref[:, 0])
    a2 = p_ref[0, :, :] + p_ref[1, :, :]
    out_ref[:, :] = a2 * nd[:, None] + b2_ref[:][None, :]


def _final(p2, deg_in, b2):
    return pl.pallas_call(
        _final_body,
        grid=(_GRID,),
        in_specs=[
            pl.BlockSpec((NC, _BLK, D), lambda i: (0, i, 0)),
            pl.BlockSpec((_BLK, 1), lambda i: (i, 0)),
            pl.BlockSpec((D,), lambda i: (0,)),
        ],
        out_specs=pl.BlockSpec((_BLK, D), lambda i: (i, 0)),
        out_shape=jax.ShapeDtypeStruct((NP, D), jnp.float32),
    )(p2, deg_in, b2)


def kernel(feat, edge_index, W1, b1, W2, b2):
    ei = edge_index.astype(jnp.int32)
    pad = jnp.full((EP - N_EDGES,), N_NODES, jnp.int32)
    srcp = jnp.concatenate([ei[0], pad])
    dstp = jnp.concatenate([ei[1], pad])
    src_w = srcp.reshape(NW, CH_W, C)
    dst_w = dstp.reshape(NW, CH_W, C)
    src_h = srcp.reshape(NW * 2, CH_W // 2, C)
    dst_h = dstp.reshape(NW * 2, CH_W // 2, C)
    featp = jnp.pad(feat, ((0, NP - N_NODES), (0, 0)))

    ident = jnp.arange(NP, dtype=jnp.int32).reshape(NS, _IDENT_CH, C)

    cnt_s = _count(src_w, ident)
    cnt_d = _count(dst_w, ident)
    deg_out = (cnt_s[0, :, 0] + cnt_s[1, :, 0]).reshape(NP, 1)
    deg_in = (cnt_d[0, :, 0] + cnt_d[1, :, 0]).reshape(NP, 1)

    x1 = _scale(deg_out, featp)
    p1 = _agg(x1, src_h, dst_h, ident)
    x2 = _mid(p1, deg_out, deg_in, W1, b1, W2)
    p2 = _agg(x2, src_h, dst_h, ident)
    outp = _final(p2, deg_in, b2)
    return outp[:N_NODES]


# speedup vs baseline: 3.6184x; 1.0012x over previous
import functools

import jax
import jax.numpy as jnp
from jax import lax
from jax.experimental import pallas as pl
from jax.experimental.pallas import tpu as pltpu
from jax.experimental.pallas import tpu_sc as plsc

N_NODES = 10000
N_EDGES = 320000
NP = 10240
EP = 327680
D = 128
HID = 256
NC = 2
NS = 16
NW = NC * NS
C = 128
CH_W = EP // NW // C
CH_T = EP // NS // C
ROWS_T = NP // NS

_mesh = plsc.VectorSubcoreMesh(core_axis_name="c", subcore_axis_name="s")


def _zero_vmem(buf, rows, cols):
    zero = jnp.zeros((16,), buf.dtype)

    @pl.loop(0, rows)
    def _(i):
        for k in range(cols // 16):
            buf[i, pl.ds(k * 16, 16)] = zero


_IDENT_CH = ROWS_T // C


@functools.partial(
    pl.kernel,
    out_type=jax.ShapeDtypeStruct((NC, NP, D), jnp.float32),
    mesh=_mesh,
    scratch_types=[
        pltpu.VMEM((CH_W, C), jnp.int32),
        pltpu.VMEM((C, D), jnp.float32),
        pltpu.VMEM((C, D), jnp.float32),
        pltpu.VMEM((_IDENT_CH, C), jnp.int32),
        pltpu.VMEM_SHARED((NP, D), jnp.float32),
        pltpu.SemaphoreType.DMA,
    ],
)
def _count(idx_hbm, ident_hbm, out_hbm, idx_v, ones_v, zbuf, ident_v, hist, sem):
    c = lax.axis_index("c")
    s = lax.axis_index("s")
    pltpu.sync_copy(idx_hbm.at[s * NC + c], idx_v)
    pltpu.sync_copy(ident_hbm.at[s], ident_v)

    one = jnp.ones((16,), jnp.float32)

    @pl.loop(0, C)
    def _(i):
        for k in range(D // 16):
            ones_v[i, pl.ds(k * 16, 16)] = one

    _zero_vmem(zbuf, C, D)
    for j in range(_IDENT_CH):
        pltpu.sync_copy(zbuf, hist.at[ident_v.at[j]])
    plsc.subcore_barrier()

    @pl.loop(0, CH_W // 8)
    def _(g):
        for k in range(8):
            pltpu.async_copy(ones_v, hist.at[idx_v.at[g * 8 + k]], sem, add=True)
        for k in range(8):
            pltpu.make_async_copy(ones_v, hist.at[idx_v.at[g * 8 + k]], sem).wait()

    plsc.subcore_barrier()
    for j in range(_IDENT_CH):
        pltpu.async_copy(hist.at[ident_v.at[j]], zbuf, sem).wait()
        pltpu.sync_copy(zbuf, out_hbm.at[c, pl.ds(s * ROWS_T + j * C, C)])


@functools.partial(
    pl.kernel,
    out_type=jax.ShapeDtypeStruct((NC, NP, D), jnp.float32),
    mesh=_mesh,
    scratch_types=[
        pltpu.VMEM((CH_W // 2, C), jnp.int32),
        pltpu.VMEM((CH_W // 2, C), jnp.int32),
        pltpu.VMEM((C, D), jnp.float32),
        pltpu.VMEM((C, D), jnp.float32),
        pltpu.VMEM((_IDENT_CH, C), jnp.int32),
        pltpu.VMEM_SHARED((NP, D), jnp.float32),
        pltpu.SemaphoreType.DMA,
        pltpu.SemaphoreType.DMA,
    ],
)
def _agg(x_hbm, src_hbm, dst_hbm, ident_hbm, out_hbm,
         src_v, dst_v, gb0, gb1, ident_v, acc, gsem, ssem):
    c = lax.axis_index("c")
    s = lax.axis_index("s")
    wid = s * NC + c
    pltpu.sync_copy(ident_hbm.at[s], ident_v)

    _zero_vmem(gb0, C, D)
    for j in range(_IDENT_CH):
        pltpu.sync_copy(gb0, acc.at[ident_v.at[j]])
    plsc.subcore_barrier()

    def g_start(j, buf):
        pltpu.async_copy(x_hbm.at[src_v.at[j]], buf, gsem)

    def g_wait(j, buf):
        pltpu.make_async_copy(x_hbm.at[src_v.at[j]], buf, gsem).wait()

    def s_start(j, buf):
        pltpu.async_copy(buf, acc.at[dst_v.at[j]], ssem, add=True)

    def s_wait(j, buf):
        pltpu.make_async_copy(buf, acc.at[dst_v.at[j]], ssem).wait()

    CH_H = CH_W // 2
    for h in range(2):
        pltpu.sync_copy(src_hbm.at[wid * 2 + h], src_v)
        pltpu.sync_copy(dst_hbm.at[wid * 2 + h], dst_v)

        g_start(0, gb0)
        g_wait(0, gb0)
        g_start(1, gb1)
        s_start(0, gb0)
        g_wait(1, gb1)
        s_wait(0, gb0)
        g_start(2, gb0)
        s_start(1, gb1)

        @pl.loop(1, CH_H // 2 - 1)
        def _(g):
            j0 = g * 2
            g_wait(j0, gb0)
            s_wait(j0 - 1, gb1)
            g_start(j0 + 1, gb1)
            s_start(j0, gb0)
            g_wait(j0 + 1, gb1)
            s_wait(j0, gb0)
            g_start(j0 + 2, gb0)
            s_start(j0 + 1, gb1)

        jl = CH_H - 2
        g_wait(jl, gb0)
        s_wait(jl - 1, gb1)
        g_start(jl + 1, gb1)
        s_start(jl, gb0)
        g_wait(jl + 1, gb1)
        s_wait(jl, gb0)
        s_start(jl + 1, gb1)
        s_wait(jl + 1, gb1)

    plsc.subcore_barrier()
    for j in range(_IDENT_CH):
        pltpu.async_copy(acc.at[ident_v.at[j]], gb0, gsem).wait()
        pltpu.sync_copy(gb0, out_hbm.at[c, pl.ds(s * ROWS_T + j * C, C)])


_BLK = 1280
_GRID = NP // _BLK


def _norm(deg):
    return jnp.where(deg > 0, lax.rsqrt(deg), 1.0)


def _scale_body(deg_out_ref, feat_ref, x1_ref):
    ns = _norm(deg_out_ref[:, 0])
    x1_ref[:, :] = feat_ref[:, :] * ns[:, None]


def _scale(deg_out, featp):
    return pl.pallas_call(
        _scale_body,
        grid=(_GRID,),
        in_specs=[
            pl.BlockSpec((_BLK, 1), lambda i: (i, 0)),
            pl.BlockSpec((_BLK, D), lambda i: (i, 0)),
        ],
        out_specs=pl.BlockSpec((_BLK, D), lambda i: (i, 0)),
        out_shape=jax.ShapeDtypeStruct((NP, D), jnp.float32),
    )(deg_out, featp)


def _mid_body(p_ref, do_ref, di_ref, w1_ref, b1_ref, w2_ref, x2_ref):
    i = pl.program_id(0)
    a1 = p_ref[0, :, :] + p_ref[1, :, :]
    nd = _norm(di_ref[:, 0])
    ns = _norm(do_ref[:, 0])
    h = jnp.dot(a1, w1_ref[:, :], preferred_element_type=jnp.float32)
    h = jnp.maximum(h * nd[:, None] + b1_ref[:][None, :], 0.0)
    x2 = jnp.dot(h * ns[:, None], w2_ref[:, :], preferred_element_type=jnp.float32)
    row = i * _BLK + lax.broadcasted_iota(jnp.int32, (_BLK, 1), 0)
    x2_ref[:, :] = jnp.where(row < N_NODES, x2, 0.0)


def _mid(p1, deg_out, deg_in, W1, b1, W2):
    return pl.pallas_call(
        _mid_body,
        grid=(_GRID,),
        in_specs=[
            pl.BlockSpec((NC, _BLK, D), lambda i: (0, i, 0)),
            pl.BlockSpec((_BLK, 1), lambda i: (i, 0)),
            pl.BlockSpec((_BLK, 1), lambda i: (i, 0)),
            pl.BlockSpec((D, HID), lambda i: (0, 0)),
            pl.BlockSpec((HID,), lambda i: (0,)),
            pl.BlockSpec((HID, D), lambda i: (0, 0)),
        ],
        out_specs=pl.BlockSpec((_BLK, D), lambda i: (i, 0)),
        out_shape=jax.ShapeDtypeStruct((NP, D), jnp.float32),
    )(p1, deg_out, deg_in, W1, b1, W2)


def _final_body(p_ref, di_ref, b2_ref, out_ref):
    nd = _norm(di_ref[:, 0])
    a2 = p_ref[0, :, :] + p_ref[1, :, :]
    out_ref[:, :] = a2 * nd[:, None] + b2_ref[:][None, :]


def _final(p2, deg_in, b2):
    return pl.pallas_call(
        _final_body,
        grid=(_GRID,),
        in_specs=[
            pl.BlockSpec((NC, _BLK, D), lambda i: (0, i, 0)),
            pl.BlockSpec((_BLK, 1), lambda i: (i, 0)),
            pl.BlockSpec((D,), lambda i: (0,)),
        ],
        out_specs=pl.BlockSpec((_BLK, D), lambda i: (i, 0)),
        out_shape=jax.ShapeDtypeStruct((NP, D), jnp.float32),
    )(p2, deg_in, b2)


def kernel(feat, edge_index, W1, b1, W2, b2):
    ei = edge_index.astype(jnp.int32)
    pad = jnp.full((EP - N_EDGES,), N_NODES, jnp.int32)
    srcp = jnp.concatenate([ei[0], pad])
    dstp = jnp.concatenate([ei[1], pad])
    src_w = srcp.reshape(NW, CH_W, C)
    dst_w = dstp.reshape(NW, CH_W, C)
    src_h = srcp.reshape(NW * 2, CH_W // 2, C)
    dst_h = dstp.reshape(NW * 2, CH_W // 2, C)
    featp = jnp.pad(feat, ((0, NP - N_NODES), (0, 0)))

    ident = jnp.arange(NP, dtype=jnp.int32).reshape(NS, _IDENT_CH, C)

    cnt_s = _count(src_w, ident)
    cnt_d = _count(dst_w, ident)
    deg_out = (cnt_s[0, :, 0] + cnt_s[1, :, 0]).reshape(NP, 1)
    deg_in = (cnt_d[0, :, 0] + cnt_d[1, :, 0]).reshape(NP, 1)

    x1 = _scale(deg_out, featp)
    p1 = _agg(x1, src_h, dst_h, ident)
    x2 = _mid(p1, deg_out, deg_in, W1, b1, W2)
    p2 = _agg(x2, src_h, dst_h, ident)
    outp = _final(p2, deg_in, b2)
    return outp[:N_NODES]
